# baseline (speedup 1.0000x reference)
import functools

import jax
import jax.numpy as jnp
from jax import lax
from jax.experimental import pallas as pl
from jax.experimental.pallas import tpu as pltpu
from jax.experimental.pallas import tpu_sc as plsc

_BATCH = 4
_SEQ = 8192
_DIM = 1024
_NUM_WORKERS = 32
_ROWS_PER_WORKER = _SEQ // _NUM_WORKERS


def _broadcast_table(pos_embedding):
    mesh = plsc.VectorSubcoreMesh(core_axis_name="c", subcore_axis_name="s")

    @functools.partial(
        pl.kernel,
        mesh=mesh,
        out_type=jax.ShapeDtypeStruct((_BATCH, _SEQ, _DIM), jnp.float32),
        scratch_types=[
            pltpu.SemaphoreType.DMA,
        ],
    )
    def k(table_hbm, out_hbm, sem):
        wid = lax.axis_index("s") * 2 + lax.axis_index("c")
        base = wid * _ROWS_PER_WORKER
        src = table_hbm.at[pl.ds(base, _ROWS_PER_WORKER)]
        cps = [
            pltpu.async_copy(
                src, out_hbm.at[b, pl.ds(base, _ROWS_PER_WORKER)], sem)
            for b in range(_BATCH)
        ]
        for cp in cps:
            cp.wait()

    return k(pos_embedding)


def kernel(input_ids, pos_embedding):
    del input_ids
    return _broadcast_table(pos_embedding)


# speedup vs baseline: 24.9455x; 24.9455x over previous
import functools

import jax
import jax.numpy as jnp
from jax import lax
from jax.experimental import pallas as pl
from jax.experimental.pallas import tpu as pltpu
from jax.experimental.pallas import tpu_sc as plsc

_BATCH = 4
_SEQ = 8192
_DIM = 1024
_NUM_WORKERS = 32
_ROWS_PER_WORKER = _SEQ // _NUM_WORKERS
_CHUNK = 64
_NCHUNKS = _ROWS_PER_WORKER // _CHUNK
_TC_BATCH = 3
_BS = 1024


def _sc_one_batch(pos_embedding):
    mesh = plsc.VectorSubcoreMesh(core_axis_name="c", subcore_axis_name="s")

    @functools.partial(
        pl.kernel,
        mesh=mesh,
        out_type=jax.ShapeDtypeStruct((1, _SEQ, _DIM), jnp.float32),
        scratch_types=[pltpu.VMEM((_CHUNK, _DIM), jnp.float32)],
    )
    def k(table_hbm, out_hbm, buf):
        wid = lax.axis_index("s") * 2 + lax.axis_index("c")
        base = wid * _ROWS_PER_WORKER
        for i in range(_NCHUNKS):
            row0 = base + i * _CHUNK
            pltpu.sync_copy(table_hbm.at[pl.ds(row0, _CHUNK)], buf)
            pltpu.sync_copy(buf, out_hbm.at[0, pl.ds(row0, _CHUNK)])

    return k(pos_embedding)


def _tc_body(in_ref, out_ref):
    row = in_ref[...]
    out_ref[...] = jnp.broadcast_to(row[None], (_TC_BATCH, _BS, _DIM))


def _tc_batches(pos_embedding):
    return pl.pallas_call(
        _tc_body,
        grid=(_SEQ // _BS,),
        in_specs=[pl.BlockSpec((_BS, _DIM), lambda i: (i, 0))],
        out_specs=pl.BlockSpec((_TC_BATCH, _BS, _DIM), lambda i: (0, i, 0)),
        out_shape=jax.ShapeDtypeStruct((_TC_BATCH, _SEQ, _DIM), jnp.float32),
    )(pos_embedding)


def kernel(input_ids, pos_embedding):
    del input_ids
    sc_part = _sc_one_batch(pos_embedding)
    tc_part = _tc_batches(pos_embedding)
    return jnp.concatenate([sc_part, tc_part], axis=0)


# speedup vs baseline: 55.6382x; 2.2304x over previous
import functools

import jax
import jax.numpy as jnp
from jax import lax
from jax.experimental import pallas as pl
from jax.experimental.pallas import tpu as pltpu
from jax.experimental.pallas import tpu_sc as plsc

_BATCH = 4
_SEQ = 8192
_DIM = 1024
_NUM_WORKERS = 32
_ROWS_PER_WORKER = _SEQ // _NUM_WORKERS
_CHUNK = 32
_NCHUNKS = _ROWS_PER_WORKER // _CHUNK
_NBUF = 3


def _broadcast_table(pos_embedding):
    mesh = plsc.VectorSubcoreMesh(core_axis_name="c", subcore_axis_name="s")

    @functools.partial(
        pl.kernel,
        mesh=mesh,
        out_type=jax.ShapeDtypeStruct((_BATCH, _SEQ, _DIM), jnp.float32),
        scratch_types=[
            pltpu.VMEM((_CHUNK, _DIM), jnp.float32),
            pltpu.VMEM((_CHUNK, _DIM), jnp.float32),
            pltpu.VMEM((_CHUNK, _DIM), jnp.float32),
            pltpu.SemaphoreType.DMA,
            pltpu.SemaphoreType.DMA,
            pltpu.SemaphoreType.DMA,
            pltpu.SemaphoreType.DMA,
            pltpu.SemaphoreType.DMA,
            pltpu.SemaphoreType.DMA,
        ],
    )
    def k(table_hbm, out_hbm, b0, b1, b2, r0, r1, r2, w0, w1, w2):
        wid = lax.axis_index("s") * 2 + lax.axis_index("c")
        base = wid * _ROWS_PER_WORKER
        bufs = (b0, b1, b2)
        rsems = (r0, r1, r2)
        wsems = (w0, w1, w2)

        def start_read(i):
            return pltpu.async_copy(
                table_hbm.at[pl.ds(base + i * _CHUNK, _CHUNK)],
                bufs[i % _NBUF], rsems[i % _NBUF])

        reads = [None] * _NCHUNKS
        writes = [[] for _ in range(_NCHUNKS)]
        reads[0] = start_read(0)
        reads[1] = start_read(1)
        for i in range(_NCHUNKS):
            p = i % _NBUF
            reads[i].wait()
            row0 = base + i * _CHUNK
            for b in range(_BATCH):
                writes[i].append(pltpu.async_copy(
                    bufs[p], out_hbm.at[b, pl.ds(row0, _CHUNK)], wsems[p]))
            if i + 2 < _NCHUNKS:
                for w in writes[i - 1]:
                    w.wait()
                reads[i + 2] = start_read(i + 2)
        for i in (_NCHUNKS - 2, _NCHUNKS - 1):
            for w in writes[i]:
                w.wait()

    return k(pos_embedding)


def kernel(input_ids, pos_embedding):
    del input_ids
    return _broadcast_table(pos_embedding)
